# baseline (speedup 1.0000x reference)
import functools

import jax
import jax.numpy as jnp
from jax.experimental import pallas as pl
from jax.experimental.pallas import tpu as pltpu

_C = 2048
_NI = 3
_NO = 3


def _norm_chunk(xb):
    s = jnp.sum(xb * xb, axis=1, keepdims=True)
    return xb * jax.lax.rsqrt(jnp.maximum(s, 1e-24))


def _body(m, x_hbm, o_hbm, ibuf, obuf, sin, sout):
    nchunks = m // _C

    def start_in(k):
        return pltpu.make_async_copy(
            x_hbm.at[pl.ds(k * _C, _C)], ibuf.at[k % _NI], sin.at[k % _NI]
        )

    def start_out(k):
        return pltpu.make_async_copy(
            obuf.at[k % _NO], o_hbm.at[pl.ds(k * _C, _C)], sout.at[k % _NO]
        )

    start_in(0).start()
    if nchunks > 1:
        start_in(1).start()
    for k in range(nchunks):
        start_in(k).wait()
        if k + 2 < nchunks:
            start_in(k + 2).start()
        if k >= _NO:
            start_out(k - _NO).wait()
        obuf[k % _NO] = _norm_chunk(ibuf[k % _NI])
        start_out(k).start()
    for k in range(max(0, nchunks - _NO), nchunks):
        start_out(k).wait()


def kernel(x):
    m, d = x.shape
    return pl.pallas_call(
        functools.partial(_body, m),
        in_specs=[pl.BlockSpec(memory_space=pltpu.MemorySpace.HBM)],
        out_specs=pl.BlockSpec(memory_space=pltpu.MemorySpace.HBM),
        out_shape=jax.ShapeDtypeStruct((m, d), x.dtype),
        scratch_shapes=[
            pltpu.VMEM((_NI, _C, d), jnp.float32),
            pltpu.VMEM((_NO, _C, d), jnp.float32),
            pltpu.SemaphoreType.DMA((_NI,)),
            pltpu.SemaphoreType.DMA((_NO,)),
        ],
    )(x)


# speedup vs baseline: 1.0474x; 1.0474x over previous
import functools

import jax
import jax.numpy as jnp
from jax.experimental import pallas as pl
from jax.experimental.pallas import tpu as pltpu

_C = 4096
_NI = 3
_NO = 3


def _norm_chunk(xb):
    s = jnp.sum(xb * xb, axis=1, keepdims=True)
    return xb * jax.lax.rsqrt(jnp.maximum(s, 1e-24))


def _body(m, x_hbm, o_hbm, ibuf, obuf, sin, sout):
    nchunks = m // _C

    def start_in(k):
        return pltpu.make_async_copy(
            x_hbm.at[pl.ds(k * _C, _C)], ibuf.at[k % _NI], sin.at[k % _NI]
        )

    def start_out(k):
        return pltpu.make_async_copy(
            obuf.at[k % _NO], o_hbm.at[pl.ds(k * _C, _C)], sout.at[k % _NO]
        )

    start_in(0).start()
    if nchunks > 1:
        start_in(1).start()
    for k in range(nchunks):
        start_in(k).wait()
        if k + 2 < nchunks:
            start_in(k + 2).start()
        if k >= _NO:
            start_out(k - _NO).wait()
        obuf[k % _NO] = _norm_chunk(ibuf[k % _NI])
        start_out(k).start()
    for k in range(max(0, nchunks - _NO), nchunks):
        start_out(k).wait()


def kernel(x):
    m, d = x.shape
    return pl.pallas_call(
        functools.partial(_body, m),
        in_specs=[pl.BlockSpec(memory_space=pltpu.MemorySpace.HBM)],
        out_specs=pl.BlockSpec(memory_space=pltpu.MemorySpace.HBM),
        out_shape=jax.ShapeDtypeStruct((m, d), x.dtype),
        scratch_shapes=[
            pltpu.VMEM((_NI, _C, d), jnp.float32),
            pltpu.VMEM((_NO, _C, d), jnp.float32),
            pltpu.SemaphoreType.DMA((_NI,)),
            pltpu.SemaphoreType.DMA((_NO,)),
        ],
    )(x)
